# Initial kernel scaffold; baseline (speedup 1.0000x reference)
import jax
import jax.numpy as jnp
from jax.experimental import pallas as pl


def kernel(x, edge_index, edge_type, norm, emb_rel, weight_neighbor, loop_weight, evolve_loop_weight, ln_scale, ln_bias):
    raise NotImplementedError("write your pallas kernel here")


# speedup vs baseline: 5.2109x; 5.2109x over previous
import functools

import jax
import jax.numpy as jnp
from jax import lax
from jax.experimental import pallas as pl
from jax.experimental.pallas import tpu as pltpu
from jax.experimental.pallas import tpu_sc as plsc

N = 10000
E = 320000
D = 128
R = 200

NC = 2
NS = 16
NW = NC * NS
EPW = E // NW
B = 80
NCHUNK = EPW // B
NPAD = 10240
ROWS_PER_TILE = NPAD // NS
DEGW = 16


def _sc_segment_sum(x, src, dst, et, emb_rel, zrow, zdeg, ones_blk, rows):
    mesh = plsc.VectorSubcoreMesh(
        core_axis_name="c", subcore_axis_name="s", num_cores=NC,
        num_subcores=NS)

    @functools.partial(
        pl.kernel,
        out_type=(
            jax.ShapeDtypeStruct((NC * NPAD, D), jnp.float32),
            jax.ShapeDtypeStruct((NC * NPAD, DEGW), jnp.float32),
        ),
        mesh=mesh,
        scratch_types=dict(
            src_v=pltpu.VMEM((B,), jnp.int32),
            dst_v=pltpu.VMEM((B,), jnp.int32),
            et_v=pltpu.VMEM((B,), jnp.int32),
            idx_v=pltpu.VMEM((B,), jnp.int32),
            xbuf=pltpu.VMEM((B, D), jnp.float32),
            rbuf=pltpu.VMEM((B, D), jnp.float32),
            dbuf=pltpu.VMEM((B, DEGW), jnp.float32),
            acc_sh=pltpu.VMEM_SHARED((NPAD, D), jnp.float32),
            deg_sh=pltpu.VMEM_SHARED((NPAD, DEGW), jnp.float32),
            sem_x=pltpu.SemaphoreType.DMA,
            sem_r=pltpu.SemaphoreType.DMA,
        ),
    )
    def sc_kernel(x_hbm, src_hbm, dst_hbm, et_hbm, rel_hbm, zrow_hbm,
                  zdeg_hbm, ones_hbm, rows_hbm, acc_out, deg_out, *,
                  src_v, dst_v, et_v, idx_v, xbuf, rbuf, dbuf, acc_sh,
                  deg_sh, sem_x, sem_r):
        cid = lax.axis_index("c")
        sid = lax.axis_index("s")
        wid = cid * NS + sid
        row0 = sid * ROWS_PER_TILE

        pltpu.sync_copy(zrow_hbm, xbuf)
        pltpu.sync_copy(zdeg_hbm, dbuf)
        for j in range(ROWS_PER_TILE // B):
            pltpu.sync_copy(rows_hbm.at[pl.ds(row0 + j * B, B)], idx_v)
            pltpu.sync_copy(xbuf, acc_sh.at[idx_v])
            pltpu.sync_copy(dbuf, deg_sh.at[idx_v])
        plsc.subcore_barrier()

        ebase = wid * EPW

        def body(i, carry):
            base = pl.multiple_of(ebase + i * B, 8)
            pltpu.sync_copy(src_hbm.at[pl.ds(base, B)], src_v)
            pltpu.sync_copy(et_hbm.at[pl.ds(base, B)], et_v)
            pltpu.sync_copy(dst_hbm.at[pl.ds(base, B)], dst_v)
            cp_x = pltpu.async_copy(x_hbm.at[src_v], xbuf, sem_x)
            cp_r = pltpu.async_copy(rel_hbm.at[et_v], rbuf, sem_r)
            cp_x.wait()
            pltpu.sync_copy(xbuf, acc_sh.at[dst_v], add=True)
            cp_r.wait()
            pltpu.sync_copy(rbuf, acc_sh.at[dst_v], add=True)
            pltpu.sync_copy(dbuf, deg_sh.at[dst_v], add=True)
            return carry

        pltpu.sync_copy(ones_hbm, dbuf)
        lax.fori_loop(0, NCHUNK, body, 0)
        plsc.subcore_barrier()

        out0 = cid * NPAD + row0
        for j in range(ROWS_PER_TILE // B):
            pltpu.sync_copy(rows_hbm.at[pl.ds(row0 + j * B, B)], idx_v)
            pltpu.sync_copy(acc_sh.at[idx_v], xbuf)
            pltpu.sync_copy(xbuf, acc_out.at[pl.ds(out0 + j * B, B)])
            pltpu.sync_copy(deg_sh.at[idx_v], dbuf)
            pltpu.sync_copy(dbuf, deg_out.at[pl.ds(out0 + j * B, B)])

    return sc_kernel(x, src, dst, et, emb_rel, zrow, zdeg, ones_blk, rows)


def _tc_tail_body(acc_ref, deg_ref, x_ref, norm_ref, wn_ref, lw_ref,
                  elw_ref, lns_ref, lnb_ref, out_ref):
    accs = acc_ref[0] + acc_ref[1]
    h = jnp.dot(accs, wn_ref[...], preferred_element_type=jnp.float32)
    h = h * norm_ref[...]
    mean = jnp.mean(h, axis=1, keepdims=True)
    var = jnp.mean(jnp.square(h - mean), axis=1, keepdims=True)
    h = (h - mean) * lax.rsqrt(var + 1e-5) * lns_ref[...] + lnb_ref[...]
    deg = deg_ref[0, :, 0:1] + deg_ref[1, :, 0:1]
    xb = x_ref[...]
    xl = jnp.dot(xb, lw_ref[...], preferred_element_type=jnp.float32)
    xe = jnp.dot(xb, elw_ref[...], preferred_element_type=jnp.float32)
    out_ref[...] = h + jnp.where(deg > 0.0, xl, xe)


def _tc_tail(acc, deg, x, norm, wn, lw, elw, lns, lnb):
    rows = 1000
    grid = (N // rows,)
    return pl.pallas_call(
        _tc_tail_body,
        grid=grid,
        in_specs=[
            pl.BlockSpec((NC, rows, D), lambda i: (0, i, 0)),
            pl.BlockSpec((NC, rows, DEGW), lambda i: (0, i, 0)),
            pl.BlockSpec((rows, D), lambda i: (i, 0)),
            pl.BlockSpec((rows, 1), lambda i: (i, 0)),
            pl.BlockSpec((D, D), lambda i: (0, 0)),
            pl.BlockSpec((D, D), lambda i: (0, 0)),
            pl.BlockSpec((D, D), lambda i: (0, 0)),
            pl.BlockSpec((1, D), lambda i: (0, 0)),
            pl.BlockSpec((1, D), lambda i: (0, 0)),
        ],
        out_specs=pl.BlockSpec((rows, D), lambda i: (i, 0)),
        out_shape=jax.ShapeDtypeStruct((N, D), jnp.float32),
    )(acc, deg, x, norm, wn, lw, elw, lns, lnb)


def kernel(x, edge_index, edge_type, norm, emb_rel, weight_neighbor,
           loop_weight, evolve_loop_weight, ln_scale, ln_bias):
    src = edge_index[0]
    dst = edge_index[1]
    zrow = jnp.zeros((B, D), jnp.float32)
    zdeg = jnp.zeros((B, DEGW), jnp.float32)
    ones_blk = jnp.ones((B, DEGW), jnp.float32)
    rows = jnp.arange(NPAD, dtype=jnp.int32)
    acc2, deg2 = _sc_segment_sum(x, src, dst, edge_type, emb_rel, zrow,
                                 zdeg, ones_blk, rows)
    acc = acc2.reshape(NC, NPAD, D)
    deg = deg2.reshape(NC, NPAD, DEGW)
    return _tc_tail(acc, deg, x, norm, weight_neighbor, loop_weight,
                    evolve_loop_weight, ln_scale.reshape(1, D),
                    ln_bias.reshape(1, D))


# speedup vs baseline: 7.2486x; 1.3910x over previous
import functools

import jax
import jax.numpy as jnp
from jax import lax
from jax.experimental import pallas as pl
from jax.experimental.pallas import tpu as pltpu
from jax.experimental.pallas import tpu_sc as plsc

N = 10000
E = 320000
D = 128
R = 200

NC = 2
NS = 16
NW = NC * NS
EPW = E // NW
B = 64
NFULL = EPW // B
TAIL = EPW - NFULL * B
NPAD = 10240
ROWS_PER_TILE = NPAD // NS
NZB = ROWS_PER_TILE // B
DEGW = 16
RPAD = 256
NSUPER = 6
SK = NFULL // NSUPER


def _sc_segment_sum(x, srcM, dstM, etM, srcT, dstT, etT, emb_rel, zrow,
                    zdeg, rows2d):
    mesh = plsc.VectorSubcoreMesh(
        core_axis_name="c", subcore_axis_name="s", num_cores=NC,
        num_subcores=NS)

    @functools.partial(
        pl.kernel,
        out_type=(
            jax.ShapeDtypeStruct((NC * NPAD, D), jnp.float32),
            jax.ShapeDtypeStruct((NW, NPAD), jnp.float32),
        ),
        mesh=mesh,
        compiler_params=pltpu.CompilerParams(needs_layout_passes=False),
        scratch_types=dict(
            srcA=pltpu.VMEM((SK, B), jnp.int32),
            dstA=pltpu.VMEM((SK, B), jnp.int32),
            etA=pltpu.VMEM((SK, B), jnp.int32),
            idx_v=pltpu.VMEM((B,), jnp.int32),
            tsrc=pltpu.VMEM((TAIL,), jnp.int32),
            tdst=pltpu.VMEM((TAIL,), jnp.int32),
            tet=pltpu.VMEM((TAIL,), jnp.int32),
            xb0=pltpu.VMEM((B, D), jnp.float32),
            xb1=pltpu.VMEM((B, D), jnp.float32),
            rb0=pltpu.VMEM((B, D), jnp.float32),
            ones_v=pltpu.VMEM((16,), jnp.float32),
            deg_local=pltpu.VMEM((NPAD,), jnp.float32),
            acc_sh=pltpu.VMEM_SHARED((NPAD, D), jnp.float32),
            sem_x0=pltpu.SemaphoreType.DMA,
            sem_x1=pltpu.SemaphoreType.DMA,
            sem_r0=pltpu.SemaphoreType.DMA,
            sem_s=pltpu.SemaphoreType.DMA,
        ),
    )
    def sc_kernel(x_hbm, srcM_hbm, dstM_hbm, etM_hbm, srcT_hbm, dstT_hbm,
                  etT_hbm, rel_hbm, zrow_hbm, zdeg_hbm, ones_hbm,
                  rows_hbm, acc_out, deg_out, *, srcA, dstA, etA, idx_v,
                  tsrc, tdst, tet, xb0, xb1, rb0, ones_v, deg_local,
                  acc_sh, sem_x0, sem_x1, sem_r0, sem_s):
        cid = lax.axis_index("c")
        sid = lax.axis_index("s")
        wid = cid * NS + sid
        row0 = sid * ROWS_PER_TILE

        pltpu.sync_copy(zrow_hbm, xb0)
        pltpu.sync_copy(zdeg_hbm, deg_local)
        for j in range(NZB):
            pltpu.sync_copy(rows_hbm.at[sid, j], idx_v)
            pltpu.sync_copy(xb0, acc_sh.at[idx_v])

        pltpu.sync_copy(srcT_hbm.at[wid], tsrc)
        pltpu.sync_copy(dstT_hbm.at[wid], tdst)
        pltpu.sync_copy(etT_hbm.at[wid], tet)
        pltpu.sync_copy(ones_hbm, ones_v)
        plsc.subcore_barrier()

        ones16 = ones_v[...]

        def count_degrees(idx_row, k):
            for g in range(B // 16):
                idx16 = idx_row[k, pl.ds(g * 16, 16)]
                plsc.addupdate_scatter(deg_local, [idx16], ones16)

        for s in range(NSUPER):
            pltpu.sync_copy(srcM_hbm.at[wid, s], srcA)
            pltpu.sync_copy(dstM_hbm.at[wid, s], dstA)
            pltpu.sync_copy(etM_hbm.at[wid, s], etA)

            @pl.loop(0, SK, step=2)
            def _(kk):
                k0 = kk
                k1 = kk + 1
                cpx0 = pltpu.async_copy(x_hbm.at[srcA.at[k0]], xb0,
                                        sem_x0)
                cpx1 = pltpu.async_copy(x_hbm.at[srcA.at[k1]], xb1,
                                        sem_x1)
                cpr0 = pltpu.async_copy(rel_hbm.at[etA.at[k0]], rb0,
                                        sem_r0)
                cpx0.wait()
                s0 = pltpu.async_copy(xb0, acc_sh.at[dstA.at[k0]], sem_s,
                                      add=True)
                cpr0.wait()
                s1 = pltpu.async_copy(rb0, acc_sh.at[dstA.at[k0]], sem_s,
                                      add=True)
                count_degrees(dstA, k0)
                s1.wait()
                cpr1 = pltpu.async_copy(rel_hbm.at[etA.at[k1]], rb0,
                                        sem_r0)
                cpx1.wait()
                s2 = pltpu.async_copy(xb1, acc_sh.at[dstA.at[k1]], sem_s,
                                      add=True)
                cpr1.wait()
                s3 = pltpu.async_copy(rb0, acc_sh.at[dstA.at[k1]], sem_s,
                                      add=True)
                count_degrees(dstA, k1)
                s0.wait()
                s2.wait()
                s3.wait()

        pltpu.async_copy(x_hbm.at[tsrc], xb0.at[pl.ds(0, TAIL)],
                         sem_x0).wait()
        pltpu.async_copy(rel_hbm.at[tet], rb0.at[pl.ds(0, TAIL)],
                         sem_r0).wait()
        pltpu.sync_copy(xb0.at[pl.ds(0, TAIL)], acc_sh.at[tdst], add=True)
        pltpu.sync_copy(rb0.at[pl.ds(0, TAIL)], acc_sh.at[tdst], add=True)
        plsc.addupdate_scatter(deg_local, [tdst[...]], ones16)
        plsc.subcore_barrier()

        out0 = cid * NPAD + row0
        for j in range(NZB):
            pltpu.sync_copy(rows_hbm.at[sid, j], idx_v)
            pltpu.sync_copy(acc_sh.at[idx_v], xb0)
            pltpu.sync_copy(xb0, acc_out.at[pl.ds(out0 + j * B, B)])
        pltpu.sync_copy(deg_local, deg_out.at[wid])

    return sc_kernel(x, srcM, dstM, etM, srcT, dstT, etT, emb_rel, zrow,
                     zdeg, jnp.ones((16,), jnp.float32), rows2d)


def _tc_tail_body(acc_ref, deg_ref, x_ref, norm_ref, wn_ref, lw_ref,
                  elw_ref, lns_ref, lnb_ref, out_ref):
    accs = acc_ref[0] + acc_ref[1]
    h = jnp.dot(accs, wn_ref[...], preferred_element_type=jnp.float32)
    h = h * norm_ref[...]
    mean = jnp.mean(h, axis=1, keepdims=True)
    var = jnp.mean(jnp.square(h - mean), axis=1, keepdims=True)
    h = (h - mean) * lax.rsqrt(var + 1e-5) * lns_ref[...] + lnb_ref[...]
    deg = jnp.sum(deg_ref[...], axis=1, keepdims=True)
    xb = x_ref[...]
    xl = jnp.dot(xb, lw_ref[...], preferred_element_type=jnp.float32)
    xe = jnp.dot(xb, elw_ref[...], preferred_element_type=jnp.float32)
    out_ref[...] = h + jnp.where(deg > 0.0, xl, xe)


def _tc_tail(acc, deg, x, norm, wn, lw, elw, lns, lnb):
    rows = 1000
    grid = (N // rows,)
    return pl.pallas_call(
        _tc_tail_body,
        grid=grid,
        in_specs=[
            pl.BlockSpec((NC, rows, D), lambda i: (0, i, 0)),
            pl.BlockSpec((rows, NW), lambda i: (i, 0)),
            pl.BlockSpec((rows, D), lambda i: (i, 0)),
            pl.BlockSpec((rows, 1), lambda i: (i, 0)),
            pl.BlockSpec((D, D), lambda i: (0, 0)),
            pl.BlockSpec((D, D), lambda i: (0, 0)),
            pl.BlockSpec((D, D), lambda i: (0, 0)),
            pl.BlockSpec((1, D), lambda i: (0, 0)),
            pl.BlockSpec((1, D), lambda i: (0, 0)),
        ],
        out_specs=pl.BlockSpec((rows, D), lambda i: (i, 0)),
        out_shape=jax.ShapeDtypeStruct((N, D), jnp.float32),
    )(acc, deg, x, norm, wn, lw, elw, lns, lnb)


def kernel(x, edge_index, edge_type, norm, emb_rel, weight_neighbor,
           loop_weight, evolve_loop_weight, ln_scale, ln_bias):
    src2d = edge_index[0].reshape(NW, EPW)
    dst2d = edge_index[1].reshape(NW, EPW)
    et2d = edge_type.reshape(NW, EPW)
    srcM = src2d[:, :NFULL * B].reshape(NW, NSUPER, SK, B)
    dstM = dst2d[:, :NFULL * B].reshape(NW, NSUPER, SK, B)
    etM = et2d[:, :NFULL * B].reshape(NW, NSUPER, SK, B)
    srcT = src2d[:, NFULL * B:]
    dstT = dst2d[:, NFULL * B:]
    etT = et2d[:, NFULL * B:]
    zrow = jnp.zeros((B, D), jnp.float32)
    zdeg = jnp.zeros((NPAD,), jnp.float32)
    rows2d = jnp.arange(NPAD, dtype=jnp.int32).reshape(NS, NZB, B)
    rel_pad = jnp.zeros((RPAD, D), jnp.float32).at[:R].set(emb_rel)
    acc2, deg2 = _sc_segment_sum(x, srcM, dstM, etM, srcT, dstT, etT,
                                 rel_pad, zrow, zdeg, rows2d)
    acc = acc2.reshape(NC, NPAD, D)
    deg_t = deg2.T
    return _tc_tail(acc, deg_t, x, norm, weight_neighbor, loop_weight,
                    evolve_loop_weight, ln_scale.reshape(1, D),
                    ln_bias.reshape(1, D))


# speedup vs baseline: 7.6006x; 1.0486x over previous
import functools

import jax
import jax.numpy as jnp
from jax import lax
from jax.experimental import pallas as pl
from jax.experimental.pallas import tpu as pltpu
from jax.experimental.pallas import tpu_sc as plsc

N = 10000
E = 320000
D = 128
R = 200

NC = 2
NS = 16
NW = NC * NS
EPW = E // NW
B = 64
NFULL = EPW // B
TAIL = EPW - NFULL * B
NPAD = 10240
ROWS_PER_TILE = NPAD // NS
NZB = ROWS_PER_TILE // B
DEGW = 16
RPAD = 256
NSUPER = 6
SK = NFULL // NSUPER


def _sc_segment_sum(x, srcM, dstM, etM, srcT, dstT, etT, emb_rel, zrow,
                    zdeg, rows2d):
    mesh = plsc.VectorSubcoreMesh(
        core_axis_name="c", subcore_axis_name="s", num_cores=NC,
        num_subcores=NS)

    @functools.partial(
        pl.kernel,
        out_type=(
            jax.ShapeDtypeStruct((NC * NPAD, D), jnp.float32),
            jax.ShapeDtypeStruct((NW, NPAD), jnp.float32),
        ),
        mesh=mesh,
        compiler_params=pltpu.CompilerParams(needs_layout_passes=False),
        scratch_types=dict(
            srcA=pltpu.VMEM((SK, B), jnp.int32),
            dstA=pltpu.VMEM((SK, B), jnp.int32),
            etA=pltpu.VMEM((SK, B), jnp.int32),
            tsrc=pltpu.VMEM((TAIL,), jnp.int32),
            tdst=pltpu.VMEM((TAIL,), jnp.int32),
            tet=pltpu.VMEM((TAIL,), jnp.int32),
            xb0=pltpu.VMEM((B, D), jnp.float32),
            xb1=pltpu.VMEM((B, D), jnp.float32),
            rb0=pltpu.VMEM((B, D), jnp.float32),
            ones_v=pltpu.VMEM((16,), jnp.float32),
            deg_local=pltpu.VMEM((NPAD,), jnp.float32),
            acc_sh=pltpu.VMEM_SHARED((NPAD, D), jnp.float32),
            sem_x0=pltpu.SemaphoreType.DMA,
            sem_x1=pltpu.SemaphoreType.DMA,
            sem_r0=pltpu.SemaphoreType.DMA,
            sem_s=pltpu.SemaphoreType.DMA,
        ),
    )
    def sc_kernel(x_hbm, srcM_hbm, dstM_hbm, etM_hbm, srcT_hbm, dstT_hbm,
                  etT_hbm, rel_hbm, zrow_hbm, zdeg_hbm, ones_hbm,
                  rows_hbm, acc_out, deg_out, *, srcA, dstA, etA,
                  tsrc, tdst, tet, xb0, xb1, rb0, ones_v, deg_local,
                  acc_sh, sem_x0, sem_x1, sem_r0, sem_s):
        cid = lax.axis_index("c")
        sid = lax.axis_index("s")
        wid = cid * NS + sid
        row0 = sid * ROWS_PER_TILE

        ci = pltpu.async_copy(rows_hbm.at[sid], srcA.at[pl.ds(0, NZB)],
                              sem_x0)
        pltpu.sync_copy(zrow_hbm, xb0)
        pltpu.sync_copy(zdeg_hbm, deg_local)
        ci.wait()
        zs = [pltpu.async_copy(xb0, acc_sh.at[srcA.at[j]], sem_s)
              for j in range(NZB)]
        for z in zs:
            z.wait()

        pltpu.sync_copy(srcT_hbm.at[wid], tsrc)
        pltpu.sync_copy(dstT_hbm.at[wid], tdst)
        pltpu.sync_copy(etT_hbm.at[wid], tet)
        pltpu.sync_copy(ones_hbm, ones_v)
        plsc.subcore_barrier()

        ones16 = ones_v[...]

        def count_degrees(idx_row, k):
            for g in range(B // 16):
                idx16 = idx_row[k, pl.ds(g * 16, 16)]
                plsc.addupdate_scatter(deg_local, [idx16], ones16)

        for s in range(NSUPER):
            ci0 = pltpu.async_copy(srcM_hbm.at[wid, s], srcA, sem_x0)
            ci1 = pltpu.async_copy(dstM_hbm.at[wid, s], dstA, sem_x1)
            ci2 = pltpu.async_copy(etM_hbm.at[wid, s], etA, sem_r0)
            ci0.wait()
            ci1.wait()
            ci2.wait()

            @pl.loop(0, SK, step=2)
            def _(kk):
                k0 = kk
                k1 = kk + 1
                cpx0 = pltpu.async_copy(x_hbm.at[srcA.at[k0]], xb0,
                                        sem_x0)
                cpx1 = pltpu.async_copy(x_hbm.at[srcA.at[k1]], xb1,
                                        sem_x1)
                cpr0 = pltpu.async_copy(rel_hbm.at[etA.at[k0]], rb0,
                                        sem_r0)
                cpx0.wait()
                s0 = pltpu.async_copy(xb0, acc_sh.at[dstA.at[k0]], sem_s,
                                      add=True)
                cpr0.wait()
                s1 = pltpu.async_copy(rb0, acc_sh.at[dstA.at[k0]], sem_s,
                                      add=True)
                count_degrees(dstA, k0)
                s0.wait()
                cpr1 = pltpu.async_copy(rel_hbm.at[etA.at[k1]], xb0,
                                        sem_r0)
                cpx1.wait()
                s2 = pltpu.async_copy(xb1, acc_sh.at[dstA.at[k1]], sem_s,
                                      add=True)
                cpr1.wait()
                s3 = pltpu.async_copy(xb0, acc_sh.at[dstA.at[k1]], sem_s,
                                      add=True)
                count_degrees(dstA, k1)
                s1.wait()
                s2.wait()
                s3.wait()

        pltpu.async_copy(x_hbm.at[tsrc], xb0.at[pl.ds(0, TAIL)],
                         sem_x0).wait()
        pltpu.async_copy(rel_hbm.at[tet], rb0.at[pl.ds(0, TAIL)],
                         sem_r0).wait()
        pltpu.sync_copy(xb0.at[pl.ds(0, TAIL)], acc_sh.at[tdst], add=True)
        pltpu.sync_copy(rb0.at[pl.ds(0, TAIL)], acc_sh.at[tdst], add=True)
        plsc.addupdate_scatter(deg_local, [tdst[...]], ones16)
        plsc.subcore_barrier()

        out0 = cid * NPAD + row0
        pltpu.sync_copy(rows_hbm.at[sid], srcA.at[pl.ds(0, NZB)])
        bufs = (xb0, xb1)
        gsems = (sem_x0, sem_x1)
        stores = []
        for j in range(NZB):
            if j >= 2:
                stores[j - 2].wait()
            g = pltpu.async_copy(acc_sh.at[srcA.at[j]], bufs[j % 2],
                                 gsems[j % 2])
            g.wait()
            stores.append(pltpu.async_copy(
                bufs[j % 2], acc_out.at[pl.ds(out0 + j * B, B)], sem_s))
        stores[NZB - 2].wait()
        stores[NZB - 1].wait()
        pltpu.sync_copy(deg_local, deg_out.at[wid])

    return sc_kernel(x, srcM, dstM, etM, srcT, dstT, etT, emb_rel, zrow,
                     zdeg, jnp.ones((16,), jnp.float32), rows2d)


def _tc_tail_body(acc_ref, deg_ref, x_ref, norm_ref, wn_ref, lw_ref,
                  elw_ref, lns_ref, lnb_ref, out_ref):
    accs = acc_ref[0] + acc_ref[1]
    h = jnp.dot(accs, wn_ref[...], preferred_element_type=jnp.float32)
    h = h * norm_ref[...]
    mean = jnp.mean(h, axis=1, keepdims=True)
    var = jnp.mean(jnp.square(h - mean), axis=1, keepdims=True)
    h = (h - mean) * lax.rsqrt(var + 1e-5) * lns_ref[...] + lnb_ref[...]
    deg = jnp.sum(deg_ref[...], axis=1, keepdims=True)
    xb = x_ref[...]
    xl = jnp.dot(xb, lw_ref[...], preferred_element_type=jnp.float32)
    xe = jnp.dot(xb, elw_ref[...], preferred_element_type=jnp.float32)
    out_ref[...] = h + jnp.where(deg > 0.0, xl, xe)


def _tc_tail(acc, deg, x, norm, wn, lw, elw, lns, lnb):
    rows = 1000
    grid = (N // rows,)
    return pl.pallas_call(
        _tc_tail_body,
        grid=grid,
        in_specs=[
            pl.BlockSpec((NC, rows, D), lambda i: (0, i, 0)),
            pl.BlockSpec((rows, NW), lambda i: (i, 0)),
            pl.BlockSpec((rows, D), lambda i: (i, 0)),
            pl.BlockSpec((rows, 1), lambda i: (i, 0)),
            pl.BlockSpec((D, D), lambda i: (0, 0)),
            pl.BlockSpec((D, D), lambda i: (0, 0)),
            pl.BlockSpec((D, D), lambda i: (0, 0)),
            pl.BlockSpec((1, D), lambda i: (0, 0)),
            pl.BlockSpec((1, D), lambda i: (0, 0)),
        ],
        out_specs=pl.BlockSpec((rows, D), lambda i: (i, 0)),
        out_shape=jax.ShapeDtypeStruct((N, D), jnp.float32),
    )(acc, deg, x, norm, wn, lw, elw, lns, lnb)


def kernel(x, edge_index, edge_type, norm, emb_rel, weight_neighbor,
           loop_weight, evolve_loop_weight, ln_scale, ln_bias):
    src2d = edge_index[0].reshape(NW, EPW)
    dst2d = edge_index[1].reshape(NW, EPW)
    et2d = edge_type.reshape(NW, EPW)
    srcM = src2d[:, :NFULL * B].reshape(NW, NSUPER, SK, B)
    dstM = dst2d[:, :NFULL * B].reshape(NW, NSUPER, SK, B)
    etM = et2d[:, :NFULL * B].reshape(NW, NSUPER, SK, B)
    srcT = src2d[:, NFULL * B:]
    dstT = dst2d[:, NFULL * B:]
    etT = et2d[:, NFULL * B:]
    zrow = jnp.zeros((B, D), jnp.float32)
    zdeg = jnp.zeros((NPAD,), jnp.float32)
    rows2d = jnp.arange(NPAD, dtype=jnp.int32).reshape(NS, NZB, B)
    rel_pad = jnp.zeros((RPAD, D), jnp.float32).at[:R].set(emb_rel)
    acc2, deg2 = _sc_segment_sum(x, srcM, dstM, etM, srcT, dstT, etT,
                                 rel_pad, zrow, zdeg, rows2d)
    acc = acc2.reshape(NC, NPAD, D)
    deg_t = deg2.T
    return _tc_tail(acc, deg_t, x, norm, weight_neighbor, loop_weight,
                    evolve_loop_weight, ln_scale.reshape(1, D),
                    ln_bias.reshape(1, D))
